# Initial kernel scaffold; baseline (speedup 1.0000x reference)
import jax
import jax.numpy as jnp
from jax.experimental import pallas as pl


def kernel(x, table):
    raise NotImplementedError("write your pallas kernel here")


# speedup vs baseline: 2.7543x; 2.7543x over previous
import functools

import jax
import jax.numpy as jnp
from jax import lax
from jax.experimental import pallas as pl
from jax.experimental.pallas import tpu as pltpu
from jax.experimental.pallas import tpu_sc as plsc

BATCH = 16384
HIST = 50
DIM = 32

_info = plsc.get_sparse_core_info()
NC, NS = _info.num_cores, _info.num_subcores
NW = NC * NS
BAGS_PER_W = BATCH // NW
CHUNK_BAGS = 32
CHUNK_IDX = CHUNK_BAGS * HIST
GATHER_SUB = 80
N_SUB = CHUNK_IDX // GATHER_SUB
N_CHUNKS = BAGS_PER_W // CHUNK_BAGS


def _ebag_kernel(x_hbm, table_hbm, out_hbm, idx_v, rows_v, out_v, sem):
    wid = lax.axis_index("s") * NC + lax.axis_index("c")

    def chunk_body(c, carry):
        idx_base = wid * (BAGS_PER_W * HIST) + c * CHUNK_IDX
        row_base = wid * BAGS_PER_W + c * CHUNK_BAGS

        pltpu.sync_copy(x_hbm.at[pl.ds(idx_base, CHUNK_IDX)], idx_v)

        copies = []
        for j in range(N_SUB):
            sl = pl.ds(j * GATHER_SUB, GATHER_SUB)
            copies.append(
                pltpu.async_copy(table_hbm.at[idx_v.at[sl]], rows_v.at[sl], sem)
            )
        for cp in copies:
            cp.wait()

        def bag_body(r, carry2):
            base = r * HIST
            a = [jnp.zeros((16,), jnp.float32) for _ in range(8)]
            for j in range(HIST):
                p = (j % 4) * 2
                a[p] = a[p] + rows_v[base + j, pl.ds(0, 16)]
                a[p + 1] = a[p + 1] + rows_v[base + j, pl.ds(16, 16)]
            s0 = (a[0] + a[2]) + (a[4] + a[6])
            s1 = (a[1] + a[3]) + (a[5] + a[7])
            scale = jnp.float32(1.0 / HIST)
            out_v[r, pl.ds(0, 16)] = s0 * scale
            out_v[r, pl.ds(16, 16)] = s1 * scale
            return carry2

        lax.fori_loop(0, CHUNK_BAGS, bag_body, 0, unroll=False)

        pltpu.sync_copy(out_v, out_hbm.at[pl.ds(row_base, CHUNK_BAGS)])
        return carry

    lax.fori_loop(0, N_CHUNKS, chunk_body, 0, unroll=False)


@jax.jit
def kernel(x, table):
    x_flat = x.reshape(-1).astype(jnp.int32)
    mesh = plsc.VectorSubcoreMesh(core_axis_name="c", subcore_axis_name="s")
    run = functools.partial(
        pl.kernel,
        mesh=mesh,
        out_type=jax.ShapeDtypeStruct((BATCH, DIM), jnp.float32),
        scratch_types=[
            pltpu.VMEM((CHUNK_IDX,), jnp.int32),
            pltpu.VMEM((CHUNK_IDX, DIM), jnp.float32),
            pltpu.VMEM((CHUNK_BAGS, DIM), jnp.float32),
            pltpu.SemaphoreType.DMA,
        ],
        compiler_params=pltpu.CompilerParams(use_tc_tiling_on_sc=False),
    )(_ebag_kernel)
    return run(x_flat, table)
